# baseline (speedup 1.0000x reference)
import functools

import jax
import jax.numpy as jnp
from jax import lax
from jax.experimental import pallas as pl
from jax.experimental.pallas import tpu as pltpu
from jax.experimental.pallas import tpu_sc as plsc

_B, _C, _H, _W = 4, 4, 384, 384
_N = _H * _W
_K = _N * 10 // 100
_ROWS = _N // 128
_NSLICES = _B * _C
_HALF = _N // 2
_HALFV = _HALF // 16


def _stage_a(no_ref, t_ref, mp_ref, res_ref, mt_ref, mpx_ref):
    no = no_ref[0, 0]
    t = t_ref[0, 0]
    res_ref[0, 0] = jnp.maximum(-(t * jnp.log(no)), 0.0)
    mt_ref[...] = jnp.full((1, 1, 1, 128), jnp.max(t), jnp.float32)
    mpx_ref[...] = jnp.full((1, 1, 1, 128), jnp.max(mp_ref[0, 0]), jnp.float32)


_SC_MESH = plsc.VectorSubcoreMesh(core_axis_name="c", subcore_axis_name="s")


@functools.partial(
    pl.kernel,
    out_type=jax.ShapeDtypeStruct((_NSLICES, 16), jnp.float32),
    mesh=_SC_MESH,
    scratch_types=[
        pltpu.VMEM((_H // 2, _W), jnp.float32),
        pltpu.VMEM((2048,), jnp.float32),
        pltpu.VMEM((128,), jnp.float32),
        pltpu.VMEM((2176,), jnp.float32),
        pltpu.VMEM((16,), jnp.float32),
        pltpu.VMEM_SHARED((16, 2176), jnp.float32),
    ],
    compiler_params=pltpu.CompilerParams(needs_layout_passes=False),
)
def _sc_topk(res_hbm, out_hbm, data, hcnt, hcrs, pbuf, loss_ref, shared):
    c = lax.axis_index("c")
    s = lax.axis_index("s")
    sl = c * 8 + lax.shift_right_logical(s, 1)
    half = jnp.bitwise_and(s, 1)
    partner = jnp.bitwise_xor(s, 1)

    pltpu.sync_copy(
        res_hbm.at[lax.shift_right_logical(sl, 2), jnp.bitwise_and(sl, 3),
                   pl.ds(half * (_H // 2), _H // 2)],
        data)

    if True:
        ones = jnp.ones((16,), jnp.float32)
        io16 = lax.broadcasted_iota(jnp.int32, (16,), 0)

        def hist_round(sh, msk, ash, prefix):
            def _z(i, carry):
                for u in range(8):
                    hcnt[pl.ds((i * 8 + u) * 16, 16)] = jnp.zeros(
                        (16,), jnp.float32)
                return carry

            lax.fori_loop(0, 2048 // 128, _z, 0)
            for u in range(128 // 16):
                hcrs[pl.ds(u * 16, 16)] = jnp.zeros((16,), jnp.float32)

            def _grp(r, g):
                b32s = [lax.bitcast_convert_type(
                    data[r, pl.ds((g + u) * 16, 16)], jnp.int32)
                    for u in range(8)]
                digs = [lax.shift_right_logical(b, sh) for b in b32s]
                if msk is not None:
                    digs = [jnp.bitwise_and(d, msk) for d in digs]
                if ash is None:
                    acts = [None] * 8
                else:
                    acts = [lax.shift_right_logical(b, ash) == prefix
                            for b in b32s]
                return digs, acts

            def _scat(grp):
                for d, a in zip(*grp):
                    plsc.addupdate_scatter(hcnt, [d], ones, mask=a)
                for d, a in zip(*grp):
                    plsc.addupdate_scatter(
                        hcrs, [lax.shift_right_logical(d, 4)], ones, mask=a)

            def _h(r, carry):
                prev = _grp(r, 0)
                for g in range(8, _W // 16, 8):
                    cur = _grp(r, g)
                    _scat(prev)
                    prev = cur
                _scat(prev)
                return carry

            lax.fori_loop(0, _H // 2, _h, 0)

            pltpu.sync_copy(hcnt, shared.at[s, pl.ds(0, 2048)])
            pltpu.sync_copy(hcrs, shared.at[s, pl.ds(2048, 128)])
            plsc.subcore_barrier()
            pltpu.sync_copy(shared.at[partner], pbuf)
            plsc.subcore_barrier()

            def _m(i, carry):
                for u in range(8):
                    j = (i * 8 + u) * 16
                    hcnt[pl.ds(j, 16)] += pbuf[pl.ds(j, 16)]
                return carry

            lax.fori_loop(0, 2048 // 128, _m, 0)
            for u in range(128 // 16):
                j = u * 16
                hcrs[pl.ds(j, 16)] += pbuf[pl.ds(2048 + j, 16)]

        def scan_round(nbins, kleft):
            ncv = nbins // 256

            def sbody(i, carry):
                done, bsel, cex, cnt_ab = carry
                idx = ncv - 1 - i
                cv = hcrs[pl.ds(idx * 16, 16)]
                tot_c = jnp.sum(cv)
                suf = lax.rev(plsc.cumsum(lax.rev(cv, (0,))), (0,))
                m = (cnt_ab + suf) >= kleft
                np_s = jnp.max(plsc.all_reduce_population_count(m))
                found = jnp.logical_and(jnp.logical_not(done), np_s >= 1)
                b_l = np_s - 1
                above = io16 > b_l
                c_excl = cnt_ab + jnp.sum(jnp.where(above, cv, 0.0))
                bsel = jnp.where(found, idx * 16 + b_l, bsel)
                cex = jnp.where(found, c_excl, cex)
                done = jnp.logical_or(done, found)
                return (done, bsel, cex, cnt_ab + tot_c)

            init = (jnp.bool_(False), jnp.int32(0), jnp.float32(0.0),
                    jnp.float32(0.0))
            _, bc, cexc, _ = lax.fori_loop(0, ncv, sbody, init)

            cv = hcnt[pl.ds(bc * 16, 16)]
            suf = lax.rev(plsc.cumsum(lax.rev(cv, (0,))), (0,))
            m = (cexc + suf) >= kleft
            np_s = jnp.max(plsc.all_reduce_population_count(m))
            b_l = np_s - 1
            cex = cexc + jnp.sum(jnp.where(io16 > b_l, cv, 0.0))
            return bc * 16 + b_l, cex

        kleft = jnp.float32(_K)

        hist_round(21, None, None, None)
        b1, cex = scan_round(1024, kleft)
        kleft = kleft - cex

        hist_round(10, 2047, 21, b1)
        b2, cex = scan_round(2048, kleft)
        kleft = kleft - cex
        p2 = lax.shift_left(b1, 11) | b2

        hist_round(0, 1023, 10, p2)
        b3, cex = scan_round(1024, kleft)
        kleft = kleft - cex

        pivot_bits = lax.shift_left(p2, 10) | b3
        pv = jnp.max(lax.bitcast_convert_type(
            jnp.full((16,), pivot_bits, jnp.int32), jnp.float32))

        z16 = jnp.zeros((16,), jnp.float32)

        def _sum_body(r, acc):
            a = list(acc)
            for k in range(_W // 16):
                v = data[r, pl.ds(k * 16, 16)]
                a[k % 4] = a[k % 4] + jnp.where(v > pv, v, 0.0)
            return tuple(a)

        accs = lax.fori_loop(0, _H // 2, _sum_body, (z16, z16, z16, z16))

        my_sum = jnp.sum(accs[0] + accs[1] + accs[2] + accs[3])
        loss_ref[...] = jnp.full((16,), my_sum, jnp.float32)
        pltpu.sync_copy(loss_ref, shared.at[s, pl.ds(0, 16)])
        plsc.subcore_barrier()
        pltpu.sync_copy(shared.at[partner, pl.ds(0, 16)], loss_ref)
        sum_gt = my_sum + jnp.max(loss_ref[...])

        loss = (sum_gt + kleft * pv) * jnp.float32(1.0 / _K)

        @pl.when(half == 0)
        def _():
            loss_ref[...] = jnp.full((16,), loss, jnp.float32)
            pltpu.sync_copy(loss_ref, out_hbm.at[sl])


@jax.jit
def kernel(net_out, target, max_positiones):
    in_spec = pl.BlockSpec((1, 1, _H, _W), lambda i, j: (i, j, 0, 0))
    res, mt, mpx = pl.pallas_call(
        _stage_a,
        grid=(_B, _C),
        in_specs=[in_spec, in_spec, in_spec],
        out_specs=[
            pl.BlockSpec((1, 1, _H, _W), lambda i, j: (i, j, 0, 0)),
            pl.BlockSpec((1, 1, 1, 128), lambda i, j: (i, j, 0, 0)),
            pl.BlockSpec((1, 1, 1, 128), lambda i, j: (i, j, 0, 0)),
        ],
        out_shape=[
            jax.ShapeDtypeStruct((_B, _C, _H, _W), jnp.float32),
            jax.ShapeDtypeStruct((_B, _C, 1, 128), jnp.float32),
            jax.ShapeDtypeStruct((_B, _C, 1, 128), jnp.float32),
        ],
    )(net_out, target, max_positiones)
    sc_loss = _sc_topk(res)[:, 0].reshape(_B, _C)
    skip = (mt[:, :, 0, 0] == 0.0) & (mpx[:, :, 0, 0] == 0.0)
    per = jnp.where(skip, 0.0, sc_loss)
    counts = jnp.count_nonzero(per, axis=1)
    img_losses = per.sum(axis=1) / counts
    return img_losses.sum() / _B


# speedup vs baseline: 1.6090x; 1.6090x over previous
import functools

import jax
import jax.numpy as jnp
from jax import lax
from jax.experimental import pallas as pl
from jax.experimental.pallas import tpu as pltpu
from jax.experimental.pallas import tpu_sc as plsc

_B, _C, _H, _W = 4, 4, 384, 384
_N = _H * _W
_K = _N * 10 // 100
_ROWS = _N // 128
_NSLICES = _B * _C
_HALF = _N // 2
_HALFV = _HALF // 16


def _stage_a(no_ref, t_ref, mp_ref, res_ref, mt_ref, mpx_ref):
    no = no_ref[0, 0]
    t = t_ref[0, 0]
    res_ref[0, 0] = jnp.maximum(-(t * jnp.log(no)), 0.0)
    mt_ref[...] = jnp.full((1, 1, 1, 128), jnp.max(t), jnp.float32)
    mpx_ref[...] = jnp.full((1, 1, 1, 128), jnp.max(mp_ref[0, 0]), jnp.float32)


_SC_MESH = plsc.VectorSubcoreMesh(core_axis_name="c", subcore_axis_name="s")


@functools.partial(
    pl.kernel,
    out_type=jax.ShapeDtypeStruct((_NSLICES, 16), jnp.float32),
    mesh=_SC_MESH,
    scratch_types=[
        pltpu.VMEM((_H // 2, _W), jnp.float32),
        pltpu.VMEM((2048,), jnp.float32),
        pltpu.VMEM((2048,), jnp.float32),
        pltpu.VMEM((16,), jnp.float32),
        pltpu.VMEM_SHARED((16, 2048), jnp.float32),
    ],
    compiler_params=pltpu.CompilerParams(needs_layout_passes=False),
)
def _sc_topk(res_hbm, out_hbm, data, hcnt, pbuf, loss_ref, shared):
    c = lax.axis_index("c")
    s = lax.axis_index("s")
    sl = c * 8 + lax.shift_right_logical(s, 1)
    half = jnp.bitwise_and(s, 1)
    partner = jnp.bitwise_xor(s, 1)

    pltpu.sync_copy(
        res_hbm.at[lax.shift_right_logical(sl, 2), jnp.bitwise_and(sl, 3),
                   pl.ds(half * (_H // 2), _H // 2)],
        data)

    ones = jnp.ones((16,), jnp.float32)
    io16 = lax.broadcasted_iota(jnp.int32, (16,), 0)

    def hist_round(sh, msk, ash, prefix):
        def _z(i, carry):
            for u in range(8):
                hcnt[pl.ds((i * 8 + u) * 16, 16)] = jnp.zeros(
                    (16,), jnp.float32)
            return carry

        lax.fori_loop(0, 2048 // 128, _z, 0)

        def _grp(r, g):
            b32s = [lax.bitcast_convert_type(
                data[r, pl.ds((g + u) * 16, 16)], jnp.int32)
                for u in range(8)]
            digs = [lax.shift_right_logical(b, sh) for b in b32s]
            if msk is not None:
                digs = [jnp.bitwise_and(d, msk) for d in digs]
            if ash is None:
                acts = [None] * 8
            else:
                acts = [lax.shift_right_logical(b, ash) == prefix
                        for b in b32s]
            return digs, acts

        def _scat(grp):
            for d, a in zip(*grp):
                plsc.addupdate_scatter(hcnt, [d], ones, mask=a)

        def _h(r, carry):
            prev = _grp(r, 0)
            for g in range(8, _W // 16, 8):
                cur = _grp(r, g)
                _scat(prev)
                prev = cur
            _scat(prev)
            return carry

        lax.fori_loop(0, _H // 2, _h, 0)

        pltpu.sync_copy(hcnt, shared.at[s])
        plsc.subcore_barrier()
        pltpu.sync_copy(shared.at[partner], pbuf)
        plsc.subcore_barrier()

        def _m(i, carry):
            for u in range(8):
                j = (i * 8 + u) * 16
                hcnt[pl.ds(j, 16)] += pbuf[pl.ds(j, 16)]
            return carry

        lax.fori_loop(0, 2048 // 128, _m, 0)

    def scan_round(nbins, kleft):
        nv = nbins // 16

        def sbody(i, carry):
            done, bsel, cex, cnt_ab = carry
            idx = nv - 1 - i
            cv = hcnt[pl.ds(idx * 16, 16)]
            tot_c = jnp.sum(cv)
            suf = lax.rev(plsc.cumsum(lax.rev(cv, (0,))), (0,))
            m = (cnt_ab + suf) >= kleft
            np_s = jnp.max(plsc.all_reduce_population_count(m))
            found = jnp.logical_and(jnp.logical_not(done), np_s >= 1)
            b_l = np_s - 1
            above = io16 > b_l
            c_excl = cnt_ab + jnp.sum(jnp.where(above, cv, 0.0))
            bsel = jnp.where(found, idx * 16 + b_l, bsel)
            cex = jnp.where(found, c_excl, cex)
            done = jnp.logical_or(done, found)
            return (done, bsel, cex, cnt_ab + tot_c)

        init = (jnp.bool_(False), jnp.int32(0), jnp.float32(0.0),
                jnp.float32(0.0))
        _, bsel, cex, _ = lax.fori_loop(0, nv, sbody, init)
        return bsel, cex

    kleft = jnp.float32(_K)

    hist_round(21, None, None, None)
    b1, cex = scan_round(1024, kleft)
    kleft = kleft - cex

    hist_round(10, 2047, 21, b1)
    b2, cex = scan_round(2048, kleft)
    kleft = kleft - cex
    p2 = lax.shift_left(b1, 11) | b2

    hist_round(0, 1023, 10, p2)
    b3, cex = scan_round(1024, kleft)
    kleft = kleft - cex

    pivot_bits = lax.shift_left(p2, 10) | b3
    pv = jnp.max(lax.bitcast_convert_type(
        jnp.full((16,), pivot_bits, jnp.int32), jnp.float32))

    z16 = jnp.zeros((16,), jnp.float32)

    def _sum_body(r, acc):
        a = list(acc)
        for k in range(_W // 16):
            v = data[r, pl.ds(k * 16, 16)]
            a[k % 4] = a[k % 4] + jnp.where(v > pv, v, 0.0)
        return tuple(a)

    accs = lax.fori_loop(0, _H // 2, _sum_body, (z16, z16, z16, z16))

    my_sum = jnp.sum(accs[0] + accs[1] + accs[2] + accs[3])
    loss_ref[...] = jnp.full((16,), my_sum, jnp.float32)
    pltpu.sync_copy(loss_ref, shared.at[s, pl.ds(0, 16)])
    plsc.subcore_barrier()
    pltpu.sync_copy(shared.at[partner, pl.ds(0, 16)], loss_ref)
    sum_gt = my_sum + jnp.max(loss_ref[...])

    loss = (sum_gt + kleft * pv) * jnp.float32(1.0 / _K)

    @pl.when(half == 0)
    def _():
        loss_ref[...] = jnp.full((16,), loss, jnp.float32)
        pltpu.sync_copy(loss_ref, out_hbm.at[sl])


@jax.jit
def kernel(net_out, target, max_positiones):
    in_spec = pl.BlockSpec((1, 1, _H, _W), lambda i, j: (i, j, 0, 0))
    res, mt, mpx = pl.pallas_call(
        _stage_a,
        grid=(_B, _C),
        in_specs=[in_spec, in_spec, in_spec],
        out_specs=[
            pl.BlockSpec((1, 1, _H, _W), lambda i, j: (i, j, 0, 0)),
            pl.BlockSpec((1, 1, 1, 128), lambda i, j: (i, j, 0, 0)),
            pl.BlockSpec((1, 1, 1, 128), lambda i, j: (i, j, 0, 0)),
        ],
        out_shape=[
            jax.ShapeDtypeStruct((_B, _C, _H, _W), jnp.float32),
            jax.ShapeDtypeStruct((_B, _C, 1, 128), jnp.float32),
            jax.ShapeDtypeStruct((_B, _C, 1, 128), jnp.float32),
        ],
    )(net_out, target, max_positiones)
    sc_loss = _sc_topk(res)[:, 0].reshape(_B, _C)
    skip = (mt[:, :, 0, 0] == 0.0) & (mpx[:, :, 0, 0] == 0.0)
    per = jnp.where(skip, 0.0, sc_loss)
    counts = jnp.count_nonzero(per, axis=1)
    img_losses = per.sum(axis=1) / counts
    return img_losses.sum() / _B
